# baseline (speedup 1.0000x reference)
import functools

import jax
import jax.numpy as jnp
from jax import lax
from jax.experimental import pallas as pl
from jax.experimental.pallas import tpu as pltpu
from jax.experimental.pallas import tpu_sc as plsc

FIELD0 = 1000000
B = 4096
IO_DIM = 2048
EMBED_DIM = 64

_NC, _NS, _L = 2, 16, 16
_NW = _NC * _NS
_BF = 2 * B
_BPW = _BF // _NW
_GCHUNK = 128

_BT = 512
_NT = B // _BT


def _sc_gather_body(xflat_hbm, table_hbm, out_hbm, idx_v, rows_v, sem):
    wid = lax.axis_index("s") * _NC + lax.axis_index("c")
    base = wid * _BPW
    pltpu.sync_copy(xflat_hbm.at[pl.ds(base, _BPW)], idx_v)
    lane = lax.iota(jnp.int32, 16)
    offs = (lane % 2) * FIELD0
    for j in range(_BPW // _L):
        sl = pl.ds(j * _L, _L)
        idx_v[sl] = idx_v[sl] + offs
    copies = []
    for j in range(_BPW // _GCHUNK):
        sl = pl.ds(j * _GCHUNK, _GCHUNK)
        copies.append(
            pltpu.async_copy(table_hbm.at[idx_v.at[sl]], rows_v.at[sl, :], sem))
    for c in copies:
        c.wait()
    pltpu.sync_copy(rows_v, out_hbm.at[pl.ds(base, _BPW)])


_sc_gather = functools.partial(
    pl.kernel,
    mesh=plsc.VectorSubcoreMesh(
        core_axis_name="c", subcore_axis_name="s",
        num_cores=_NC, num_subcores=_NS),
    out_type=jax.ShapeDtypeStruct((_BF, EMBED_DIM), jnp.float32),
    scratch_types=[
        pltpu.VMEM((_BPW,), jnp.int32),
        pltpu.VMEM((_BPW, EMBED_DIM), jnp.float32),
        pltpu.SemaphoreType.DMA,
    ],
    compiler_params=pltpu.CompilerParams(use_tc_tiling_on_sc=False),
)(_sc_gather_body)


def _tc_body(ux_ref, ix_ref, emb_ref, ueW_ref, ueb_ref, ieW_ref, ieb_ref,
             W1_ref, b1_ref, g1_ref, be1_ref, W2_ref, b2_ref, g2_ref, be2_ref,
             W3_ref, b3_ref, out_ref, h1_scr):
    t = pl.program_id(0)
    uz = jnp.maximum(
        jnp.dot(ux_ref[...], ueW_ref[...], preferred_element_type=jnp.float32)
        + ueb_ref[...], 0.0)
    iz = jnp.maximum(
        jnp.dot(ix_ref[...], ieW_ref[...], preferred_element_type=jnp.float32)
        + ieb_ref[...], 0.0)
    emb = emb_ref[...]
    h1pre = (
        jnp.dot(uz, W1_ref[0:64, :], preferred_element_type=jnp.float32)
        + jnp.dot(emb[:, 0:64], W1_ref[64:128, :],
                  preferred_element_type=jnp.float32)
        + jnp.dot(iz, W1_ref[128:192, :], preferred_element_type=jnp.float32)
        + jnp.dot(emb[:, 64:128], W1_ref[192:256, :],
                  preferred_element_type=jnp.float32)
        + b1_ref[...])
    h1_scr[pl.ds(t * _BT, _BT), :] = h1pre

    @pl.when(t == _NT - 1)
    def _finish():
        hp = h1_scr[...]
        m1 = jnp.mean(hp, axis=0, keepdims=True)
        v1 = jnp.mean((hp - m1) ** 2, axis=0, keepdims=True)
        h1 = jnp.maximum(
            g1_ref[...] * (hp - m1) * lax.rsqrt(v1 + 1e-5) + be1_ref[...], 0.0)
        h2pre = (jnp.dot(h1, W2_ref[...], preferred_element_type=jnp.float32)
                 + b2_ref[...])
        m2 = jnp.mean(h2pre, axis=0, keepdims=True)
        v2 = jnp.mean((h2pre - m2) ** 2, axis=0, keepdims=True)
        h2 = jnp.maximum(
            g2_ref[...] * (h2pre - m2) * lax.rsqrt(v2 + 1e-5) + be2_ref[...],
            0.0)
        y = jnp.maximum(
            jnp.dot(h2, W3_ref[...], preferred_element_type=jnp.float32)
            + b3_ref[...], 0.0)
        out_ref[...] = y


def _tc_call(user_x, item_x, emb, ue_W, ue_b, ie_W, ie_b,
             W1, b1, g1, be1, W2, b2, g2, be2, W3, b3):
    full = lambda shape: pl.BlockSpec(shape, lambda t: (0, 0))
    return pl.pallas_call(
        _tc_body,
        grid=(_NT,),
        in_specs=[
            pl.BlockSpec((_BT, IO_DIM), lambda t: (t, 0)),
            pl.BlockSpec((_BT, IO_DIM), lambda t: (t, 0)),
            pl.BlockSpec((_BT, 2 * EMBED_DIM), lambda t: (t, 0)),
            full((IO_DIM, 64)), full((1, 64)),
            full((IO_DIM, 64)), full((1, 64)),
            full((256, 256)), full((1, 256)), full((1, 256)), full((1, 256)),
            full((256, 128)), full((1, 128)), full((1, 128)), full((1, 128)),
            full((128, 1)), full((1, 1)),
        ],
        out_specs=pl.BlockSpec((B, 1), lambda t: (0, 0)),
        out_shape=jax.ShapeDtypeStruct((B, 1), jnp.float32),
        scratch_shapes=[pltpu.VMEM((B, 256), jnp.float32)],
    )(user_x, item_x, emb, ue_W, ue_b, ie_W, ie_b,
      W1, b1, g1, be1, W2, b2, g2, be2, W3, b3)


def kernel(x, user_x, item_x, emb_table, ue_W, ue_b, ie_W, ie_b,
           W1, b1, g1, be1, W2, b2, g2, be2, W3, b3):
    xflat = x.reshape(_BF).astype(jnp.int32)
    rows = _sc_gather(xflat, emb_table)
    emb = rows.reshape(B, 2 * EMBED_DIM)
    r2 = lambda a: a.reshape(1, -1)
    return _tc_call(user_x, item_x, emb, ue_W, r2(ue_b), ie_W, r2(ie_b),
                    W1, r2(b1), r2(g1), r2(be1),
                    W2, r2(b2), r2(g2), r2(be2),
                    W3, r2(b3))
